# baseline (speedup 1.0000x reference)
import functools

import numpy as np
import jax
import jax.numpy as jnp
from jax.experimental import pallas as pl
from jax.experimental.pallas import tpu as pltpu

_ALPHA = 4


def _slow_gather_dma_kernel(idx, frames_ref, slow_ref, sem):
    copies = []
    for c in range(frames_ref.shape[0]):
        for t, s in enumerate(idx):
            copies.append(
                pltpu.make_async_copy(frames_ref.at[c, s], slow_ref.at[c, t], sem)
            )
    for cp in copies:
        cp.start()
    for cp in copies:
        cp.wait()


def kernel(frames):
    C, T, H, W = frames.shape
    Ts = T // _ALPHA
    idx = [int(i) for i in np.linspace(0.0, T - 1, Ts).astype(np.int32)]
    slow = pl.pallas_call(
        functools.partial(_slow_gather_dma_kernel, idx),
        out_shape=jax.ShapeDtypeStruct((C, Ts, H, W), frames.dtype),
        in_specs=[pl.BlockSpec(memory_space=pltpu.HBM)],
        out_specs=pl.BlockSpec(memory_space=pltpu.HBM),
        scratch_shapes=[pltpu.SemaphoreType.DMA],
    )(frames)
    return (slow, frames)


# speedup vs baseline: 9.4661x; 9.4661x over previous
import numpy as np
import jax
import jax.numpy as jnp
from jax.experimental import pallas as pl
from jax.experimental.pallas import tpu as pltpu

_ALPHA = 4


def _row_copy_kernel(idx_ref, src_ref, dst_ref):
    dst_ref[...] = src_ref[...]


def kernel(frames):
    C, T, H, W = frames.shape
    Ts = T // _ALPHA
    idx = np.linspace(0.0, T - 1, Ts).astype(np.int32)
    src_rows = (np.arange(C)[:, None] * T + idx[None, :]).reshape(-1).astype(np.int32)

    grid_spec = pltpu.PrefetchScalarGridSpec(
        num_scalar_prefetch=1,
        grid=(C * Ts,),
        in_specs=[
            pl.BlockSpec((1, H, W), lambda i, idx_ref: (idx_ref[i], 0, 0)),
        ],
        out_specs=pl.BlockSpec((1, H, W), lambda i, idx_ref: (i, 0, 0)),
    )
    slow = pl.pallas_call(
        _row_copy_kernel,
        grid_spec=grid_spec,
        out_shape=jax.ShapeDtypeStruct((C * Ts, H, W), frames.dtype),
    )(jnp.asarray(src_rows), frames.reshape(C * T, H, W))
    return (slow.reshape(C, Ts, H, W), frames)
